# Initial kernel scaffold; baseline (speedup 1.0000x reference)
import jax
import jax.numpy as jnp
from jax.experimental import pallas as pl


def kernel(x, edge_index, W1_rel, b1_rel, W1_root, W2_rel, b2_rel, W2_root):
    raise NotImplementedError("write your pallas kernel here")


# speedup vs baseline: 4.9235x; 4.9235x over previous
import functools

import jax
import jax.numpy as jnp
from jax import lax
from jax.experimental import pallas as pl
from jax.experimental.pallas import tpu as pltpu
from jax.experimental.pallas import tpu_sc as plsc

N = 10000
E = 320000
D = 128

NC = 2
NS = 16
NW = NC * NS
C = 128
K = -(-E // (NW * C))
EPAD = NW * K * C
NPAD = 10112
RZ = NPAD // NS


def _segment_sum_sc(x, srcs, dsts, zeros):
  mesh = plsc.VectorSubcoreMesh(core_axis_name="c", subcore_axis_name="s")

  @functools.partial(
      pl.kernel,
      mesh=mesh,
      out_type=jax.ShapeDtypeStruct((NC, NPAD, D), jnp.float32),
      scratch_types=[
          pltpu.VMEM((K, C), jnp.int32),
          pltpu.VMEM((K, C), jnp.int32),
          pltpu.VMEM((C, D), jnp.float32),
          pltpu.VMEM_SHARED((NPAD, D), jnp.float32),
          pltpu.SemaphoreType.DMA,
      ],
  )
  def seg_kernel(x_hbm, src_hbm, dst_hbm, zero_hbm, out_hbm,
                 src_v, dst_v, rows_v, acc_sh, sem):
    cid = lax.axis_index("c")
    sid = lax.axis_index("s")
    wid = sid * NC + cid

    pltpu.sync_copy(zero_hbm.at[pl.ds(sid * RZ, RZ)],
                    acc_sh.at[pl.ds(sid * RZ, RZ)])
    pltpu.sync_copy(src_hbm.at[wid], src_v)
    pltpu.sync_copy(dst_hbm.at[wid], dst_v)
    plsc.subcore_barrier()

    @pl.loop(0, K)
    def _(k):
      pltpu.async_copy(x_hbm.at[src_v.at[k]], rows_v, sem).wait()
      pltpu.sync_copy(rows_v, acc_sh.at[dst_v.at[k]], add=True)

    plsc.subcore_barrier()
    pltpu.sync_copy(acc_sh.at[pl.ds(sid * RZ, RZ)],
                    out_hbm.at[cid].at[pl.ds(sid * RZ, RZ)])

  return seg_kernel(x, srcs, dsts, zeros)


BN = 1000


def _combine_tc(parts, x, w_rel, b_rel, w_root):
  def body(p0_ref, p1_ref, x_ref, wrel_ref, wroot_ref, b_ref, o_ref):
    agg = p0_ref[0] + p1_ref[0]
    dn = (((1,), (1,)), ((), ()))
    rel = lax.dot_general(agg, wrel_ref[...], dn,
                          preferred_element_type=jnp.float32)
    root = lax.dot_general(x_ref[...], wroot_ref[...], dn,
                           preferred_element_type=jnp.float32)
    o_ref[...] = rel + root + b_ref[...]

  return pl.pallas_call(
      body,
      grid=(N // BN,),
      in_specs=[
          pl.BlockSpec((1, BN, D), lambda i: (0, i, 0)),
          pl.BlockSpec((1, BN, D), lambda i: (1, i, 0)),
          pl.BlockSpec((BN, D), lambda i: (i, 0)),
          pl.BlockSpec((D, D), lambda i: (0, 0)),
          pl.BlockSpec((D, D), lambda i: (0, 0)),
          pl.BlockSpec((D,), lambda i: (0,)),
      ],
      out_specs=pl.BlockSpec((BN, D), lambda i: (i, 0)),
      out_shape=jax.ShapeDtypeStruct((N, D), jnp.float32),
  )(parts, parts, x, w_rel, w_root, b_rel)


def kernel(x, edge_index, W1_rel, b1_rel, W1_root, W2_rel, b2_rel, W2_root):
  src = edge_index[0]
  dst = edge_index[1]
  pad = EPAD - E
  srcs = jnp.concatenate([src, jnp.zeros((pad,), jnp.int32)])
  dsts = jnp.concatenate([dst, jnp.full((pad,), N, jnp.int32)])
  srcs = srcs.reshape(NW, K, C)
  dsts = dsts.reshape(NW, K, C)
  zeros = jnp.zeros((NPAD, D), jnp.float32)

  p1 = _segment_sum_sc(x, srcs, dsts, zeros)
  h = _combine_tc(p1, x, W1_rel, b1_rel, W1_root)
  p2 = _segment_sum_sc(h, srcs, dsts, zeros)
  return _combine_tc(p2, h, W2_rel, b2_rel, W2_root)
